# baseline (speedup 1.0000x reference)
import jax
import jax.numpy as jnp
from jax import lax
from jax.experimental import pallas as pl
from jax.experimental.pallas import tpu as pltpu
from jax.experimental.pallas import tpu_sc as plsc

N_NODES = 10000
D = 128
N_EDGES = 320000

NUM_CORES = 2
NUM_SUBCORES = 16
NUM_WORKERS = NUM_CORES * NUM_SUBCORES
CHUNK = 128
CHUNKS_PER_WORKER = 80
CHUNKS_PER_STAGE = 8
E_PAD = NUM_WORKERS * CHUNKS_PER_WORKER * CHUNK

ACC_ROWS = 10240
DUMMY_ROW = N_NODES
ROWS_PER_TILE = ACC_ROWS // NUM_SUBCORES
ZROWS = 128


def _sc_body(x_hbm, edges_hbm, out_hbm, ei_a, ei_b, rows_a, rows_b,
             acc, sem_a, sem_b, sem_ia, sem_ib):
    c = lax.axis_index("c")
    s = lax.axis_index("s")
    wid = s * NUM_CORES + c
    x2 = x_hbm.at[0].at[0]

    zeros16 = jnp.zeros((16,), jnp.float32)

    @pl.loop(0, ZROWS)
    def _zero_row(r):
        for j in range(D // 16):
            rows_a[r, pl.ds(j * 16, 16)] = zeros16

    for j in range(ROWS_PER_TILE // ZROWS):
        pltpu.sync_copy(rows_a.at[pl.ds(0, ZROWS)],
                        acc.at[pl.ds(s * ROWS_PER_TILE + j * ZROWS, ZROWS)])
    plsc.subcore_barrier()

    n_stages = CHUNKS_PER_WORKER // CHUNKS_PER_STAGE
    last = CHUNKS_PER_STAGE - 2

    def _stage_base(stage):
        return wid * CHUNKS_PER_WORKER + stage * CHUNKS_PER_STAGE

    def _idx_copy(stage, ei, sem):
        return pltpu.make_async_copy(
            edges_hbm.at[pl.ds(0, 2),
                         pl.ds(_stage_base(stage), CHUNKS_PER_STAGE)],
            ei, sem)

    _idx_copy(0, ei_a, sem_ia).start()
    for stage in range(n_stages):
        even = stage % 2 == 0
        ei, sem = (ei_a, sem_ia) if even else (ei_b, sem_ib)
        _idx_copy(stage, ei, sem).wait()
        if stage + 1 < n_stages:
            _idx_copy(stage + 1, ei_b if even else ei_a,
                      sem_ib if even else sem_ia).start()
        pltpu.async_copy(x2.at[ei.at[0, 0]], rows_a, sem_a)

        @pl.loop(0, last, step=2)
        def _edges(k):
            pltpu.make_async_copy(x2.at[ei.at[0, k]], rows_a, sem_a).wait()
            pltpu.async_copy(x2.at[ei.at[0, k + 1]], rows_b, sem_b)
            pltpu.sync_copy(rows_a, acc.at[ei.at[1, k]], add=True)
            pltpu.make_async_copy(x2.at[ei.at[0, k + 1]], rows_b,
                                  sem_b).wait()
            pltpu.async_copy(x2.at[ei.at[0, k + 2]], rows_a, sem_a)
            pltpu.sync_copy(rows_b, acc.at[ei.at[1, k + 1]], add=True)

        pltpu.make_async_copy(x2.at[ei.at[0, last]], rows_a, sem_a).wait()
        pltpu.async_copy(x2.at[ei.at[0, last + 1]], rows_b, sem_b)
        pltpu.sync_copy(rows_a, acc.at[ei.at[1, last]], add=True)
        pltpu.make_async_copy(x2.at[ei.at[0, last + 1]], rows_b,
                              sem_b).wait()
        pltpu.sync_copy(rows_b, acc.at[ei.at[1, last + 1]], add=True)

    plsc.subcore_barrier()

    pltpu.sync_copy(
        acc.at[pl.ds(s * ROWS_PER_TILE, ROWS_PER_TILE)],
        out_hbm.at[c, pl.ds(s * ROWS_PER_TILE, ROWS_PER_TILE)])


_sc_scatter = pl.kernel(
    _sc_body,
    out_type=jax.ShapeDtypeStruct((NUM_CORES, ACC_ROWS, D), jnp.float32),
    mesh=plsc.VectorSubcoreMesh(core_axis_name="c", subcore_axis_name="s"),
    scratch_types=[
        pltpu.VMEM((2, CHUNKS_PER_STAGE, CHUNK), jnp.int32),
        pltpu.VMEM((2, CHUNKS_PER_STAGE, CHUNK), jnp.int32),
        pltpu.VMEM((CHUNK, D), jnp.float32),
        pltpu.VMEM((CHUNK, D), jnp.float32),
        pltpu.VMEM_SHARED((ACC_ROWS, D), jnp.float32),
        pltpu.SemaphoreType.DMA,
        pltpu.SemaphoreType.DMA,
        pltpu.SemaphoreType.DMA,
        pltpu.SemaphoreType.DMA,
    ],
)

_CB = 1000


def _combine_body(a_ref, b_ref, o_ref):
    o_ref[...] = a_ref[0] + b_ref[0]


_combine = pl.pallas_call(
    _combine_body,
    grid=(N_NODES // _CB,),
    in_specs=[
        pl.BlockSpec((1, _CB, D), lambda i: (0, i, 0)),
        pl.BlockSpec((1, _CB, D), lambda i: (1, i, 0)),
    ],
    out_specs=pl.BlockSpec((_CB, D), lambda i: (i, 0)),
    out_shape=jax.ShapeDtypeStruct((N_NODES, D), jnp.float32),
)


@jax.jit
def kernel(x, edge_index):
    pad = E_PAD - N_EDGES
    pad_src = jnp.arange(pad, dtype=jnp.int32) % N_NODES
    pad_dst = DUMMY_ROW + jnp.arange(pad, dtype=jnp.int32) % (ACC_ROWS
                                                              - N_NODES)
    edges = jnp.concatenate(
        [edge_index, jnp.stack([pad_src, pad_dst])], axis=1,
    ).reshape(2, E_PAD // CHUNK, CHUNK)
    partials = _sc_scatter(x, edges)
    out = _combine(partials, partials)
    return out.reshape(x.shape)


# speedup vs baseline: 1.0283x; 1.0283x over previous
import jax
import jax.numpy as jnp
from jax import lax
from jax.experimental import pallas as pl
from jax.experimental.pallas import tpu as pltpu
from jax.experimental.pallas import tpu_sc as plsc

N_NODES = 10000
D = 128
N_EDGES = 320000

NUM_CORES = 2
NUM_SUBCORES = 16
NUM_WORKERS = NUM_CORES * NUM_SUBCORES
CHUNK = 125
CHUNKS_PER_WORKER = N_EDGES // (NUM_WORKERS * CHUNK)
CHUNKS_PER_STAGE = 16

ACC_ROWS = 10240
ROWS_PER_TILE = ACC_ROWS // NUM_SUBCORES
ZROWS = 80


def _sc_body(x_hbm, edges_hbm, out_hbm, ei_a, ei_b, rows_a, rows_b,
             acc, sem_a, sem_b, sem_ia, sem_ib):
    c = lax.axis_index("c")
    s = lax.axis_index("s")
    wid = s * NUM_CORES + c
    x2 = x_hbm.at[0].at[0]

    def _stage_base(stage):
        return wid * CHUNKS_PER_WORKER + stage * CHUNKS_PER_STAGE

    def _idx_copy(stage, ei, sem):
        return pltpu.make_async_copy(
            edges_hbm.at[pl.ds(0, 2),
                         pl.ds(_stage_base(stage), CHUNKS_PER_STAGE)],
            ei, sem)

    _idx_copy(0, ei_a, sem_ia).start()
    zeros16 = jnp.zeros((16,), jnp.float32)

    @pl.loop(0, ZROWS)
    def _zero_row(r):
        for j in range(D // 16):
            rows_a[r, pl.ds(j * 16, 16)] = zeros16

    for j in range(ROWS_PER_TILE // ZROWS):
        pltpu.sync_copy(rows_a.at[pl.ds(0, ZROWS)],
                        acc.at[pl.ds(s * ROWS_PER_TILE + j * ZROWS, ZROWS)])
    plsc.subcore_barrier()

    n_stages = CHUNKS_PER_WORKER // CHUNKS_PER_STAGE
    last = CHUNKS_PER_STAGE - 2

    for stage in range(n_stages):
        even = stage % 2 == 0
        ei, sem = (ei_a, sem_ia) if even else (ei_b, sem_ib)
        _idx_copy(stage, ei, sem).wait()
        if stage + 1 < n_stages:
            _idx_copy(stage + 1, ei_b if even else ei_a,
                      sem_ib if even else sem_ia).start()
        pltpu.async_copy(x2.at[ei.at[0, 0]], rows_a, sem_a)

        @pl.loop(0, last, step=2)
        def _edges(k):
            pltpu.make_async_copy(x2.at[ei.at[0, k]], rows_a, sem_a).wait()
            pltpu.async_copy(x2.at[ei.at[0, k + 1]], rows_b, sem_b)
            pltpu.sync_copy(rows_a, acc.at[ei.at[1, k]], add=True)
            pltpu.make_async_copy(x2.at[ei.at[0, k + 1]], rows_b,
                                  sem_b).wait()
            pltpu.async_copy(x2.at[ei.at[0, k + 2]], rows_a, sem_a)
            pltpu.sync_copy(rows_b, acc.at[ei.at[1, k + 1]], add=True)

        pltpu.make_async_copy(x2.at[ei.at[0, last]], rows_a, sem_a).wait()
        pltpu.async_copy(x2.at[ei.at[0, last + 1]], rows_b, sem_b)
        pltpu.sync_copy(rows_a, acc.at[ei.at[1, last]], add=True)
        pltpu.make_async_copy(x2.at[ei.at[0, last + 1]], rows_b,
                              sem_b).wait()
        pltpu.sync_copy(rows_b, acc.at[ei.at[1, last + 1]], add=True)

    plsc.subcore_barrier()

    pltpu.sync_copy(
        acc.at[pl.ds(s * ROWS_PER_TILE, ROWS_PER_TILE)],
        out_hbm.at[c, pl.ds(s * ROWS_PER_TILE, ROWS_PER_TILE)])


_sc_scatter = pl.kernel(
    _sc_body,
    out_type=jax.ShapeDtypeStruct((NUM_CORES, ACC_ROWS, D), jnp.float32),
    mesh=plsc.VectorSubcoreMesh(core_axis_name="c", subcore_axis_name="s"),
    scratch_types=[
        pltpu.VMEM((2, CHUNKS_PER_STAGE, CHUNK), jnp.int32),
        pltpu.VMEM((2, CHUNKS_PER_STAGE, CHUNK), jnp.int32),
        pltpu.VMEM((CHUNK, D), jnp.float32),
        pltpu.VMEM((CHUNK, D), jnp.float32),
        pltpu.VMEM_SHARED((ACC_ROWS, D), jnp.float32),
        pltpu.SemaphoreType.DMA,
        pltpu.SemaphoreType.DMA,
        pltpu.SemaphoreType.DMA,
        pltpu.SemaphoreType.DMA,
    ],
)

_CB = 1000


def _combine_body(a_ref, b_ref, o_ref):
    o_ref[...] = a_ref[0] + b_ref[0]


_combine = pl.pallas_call(
    _combine_body,
    grid=(N_NODES // _CB,),
    in_specs=[
        pl.BlockSpec((1, _CB, D), lambda i: (0, i, 0)),
        pl.BlockSpec((1, _CB, D), lambda i: (1, i, 0)),
    ],
    out_specs=pl.BlockSpec((_CB, D), lambda i: (i, 0)),
    out_shape=jax.ShapeDtypeStruct((N_NODES, D), jnp.float32),
)


@jax.jit
def kernel(x, edge_index):
    edges = edge_index.reshape(2, N_EDGES // CHUNK, CHUNK)
    partials = _sc_scatter(x, edges)
    out = _combine(partials, partials)
    return out.reshape(x.shape)
